# baseline (speedup 1.0000x reference)
import functools

import jax
import jax.numpy as jnp
from jax import lax
from jax.experimental import pallas as pl
from jax.experimental.pallas import tpu as pltpu
from jax.experimental.pallas import tpu_sc as plsc

_LANES = 128
_G = 2
_CHUNK = _G * _LANES
_NBUF = 2


@functools.partial(jax.jit, static_argnames=("n_pairs",))
def _gather_pairs(xe, xo, t2, n_pairs):
    info = plsc.get_sparse_core_info()
    nw = info.num_cores * info.num_subcores
    per_w = n_pairs // nw
    steps = per_w // _CHUNK
    mesh = plsc.VectorSubcoreMesh(core_axis_name="c", subcore_axis_name="s")

    @functools.partial(
        pl.kernel,
        mesh=mesh,
        out_type=jax.ShapeDtypeStruct((n_pairs, 2 * 64), jnp.float32),
        scratch_types=[
            pltpu.VMEM((_CHUNK,), jnp.int32),
            pltpu.VMEM((_CHUNK,), jnp.int32),
            pltpu.VMEM((_G, _LANES), jnp.int32),
            pltpu.VMEM((_NBUF, _CHUNK, 2 * 64), jnp.float32),
            pltpu.VMEM_SHARED((100, 2 * 64), jnp.float32),
            pltpu.SemaphoreType.DMA,
            pltpu.SemaphoreType.DMA,
            pltpu.SemaphoreType.DMA,
        ],
    )
    def k(xe_hbm, xo_hbm, t2_hbm, out_hbm, xe_v, xo_v, pair_v, rows_v,
          t2_sh, sem_g, sem_w0, sem_w1):
        sid = lax.axis_index("s")
        wid = sid * info.num_cores + lax.axis_index("c")
        sem_w = (sem_w0, sem_w1)

        @pl.when(sid == 0)
        def _():
            pltpu.sync_copy(t2_hbm, t2_sh)

        plsc.subcore_barrier()

        def outer(io, carry):
            for b in range(_NBUF):
                i = _NBUF * io + b
                p0 = wid * per_w + i * _CHUNK
                pltpu.sync_copy(xe_hbm.at[pl.ds(p0, _CHUNK)], xe_v)
                pltpu.sync_copy(xo_hbm.at[pl.ds(p0, _CHUNK)], xo_v)
                for t in range(_CHUNK // 16):
                    e = xe_v[pl.ds(t * 16, 16)]
                    o = xo_v[pl.ds(t * 16, 16)]
                    pair_v[t // 8, pl.ds((t % 8) * 16, 16)] = e * 10 + o

                @pl.when(io >= 1)
                def _():
                    pltpu.make_async_copy(
                        rows_v.at[b], out_hbm.at[pl.ds(0, _CHUNK)], sem_w[b]
                    ).wait()

                copies = [
                    pltpu.async_copy(
                        t2_sh.at[pair_v.at[j]],
                        rows_v.at[b, pl.ds(j * _LANES, _LANES)],
                        sem_g,
                    )
                    for j in range(_G)
                ]
                for c in copies:
                    c.wait()
                pltpu.async_copy(
                    rows_v.at[b], out_hbm.at[pl.ds(p0, _CHUNK)], sem_w[b]
                )
            return carry

        lax.fori_loop(0, steps // _NBUF, outer, 0)
        for b in range(_NBUF):
            pltpu.make_async_copy(
                rows_v.at[b], out_hbm.at[pl.ds(0, _CHUNK)], sem_w[b]
            ).wait()

    return k(xe, xo, t2)


def kernel(x, table):
    b, s = x.shape
    v, d = table.shape
    n = b * s
    t2 = jnp.concatenate(
        [
            jnp.broadcast_to(table[:, None, :], (v, v, d)),
            jnp.broadcast_to(table[None, :, :], (v, v, d)),
        ],
        axis=-1,
    ).reshape(v * v, 2 * d)
    xp = x.reshape(n // 2, 2)
    out = _gather_pairs(xp[:, 0], xp[:, 1], t2, n // 2)
    return out.reshape(b, s, d)


# speedup vs baseline: 1.4824x; 1.4824x over previous
import functools

import jax
import jax.numpy as jnp
from jax import lax
from jax.experimental import pallas as pl
from jax.experimental.pallas import tpu as pltpu
from jax.experimental.pallas import tpu_sc as plsc

_TAKE_DN = lax.GatherDimensionNumbers(
    offset_dims=(), collapsed_slice_dims=(0,), start_index_map=(0,)
)


def _take16(v, idx):
    return lax.gather(
        v, idx[:, None], _TAKE_DN, slice_sizes=(1,),
        mode=lax.GatherScatterMode.PROMISE_IN_BOUNDS,
    )


_LANES = 128
_G = 2
_CHUNK = _G * _LANES
_NBUF = 2


@functools.partial(jax.jit, static_argnames=("n_pairs",))
def _gather_pairs(xf, t2, n_pairs):
    info = plsc.get_sparse_core_info()
    nw = info.num_cores * info.num_subcores
    per_w = n_pairs // nw
    steps = per_w // _CHUNK
    mesh = plsc.VectorSubcoreMesh(core_axis_name="c", subcore_axis_name="s")

    @functools.partial(
        pl.kernel,
        mesh=mesh,
        out_type=jax.ShapeDtypeStruct((n_pairs, 2 * 64), jnp.float32),
        scratch_types=[
            pltpu.VMEM((2 * _CHUNK,), jnp.int32),
            pltpu.VMEM((_G, _LANES), jnp.int32),
            pltpu.VMEM((_NBUF, _CHUNK, 2 * 64), jnp.float32),
            pltpu.VMEM_SHARED((100, 2 * 64), jnp.float32),
            pltpu.SemaphoreType.DMA,
            pltpu.SemaphoreType.DMA,
            pltpu.SemaphoreType.DMA,
        ],
    )
    def k(xf_hbm, t2_hbm, out_hbm, raw_v, pair_v, rows_v,
          t2_sh, sem_g, sem_w0, sem_w1):
        sid = lax.axis_index("s")
        wid = sid * info.num_cores + lax.axis_index("c")
        sem_w = (sem_w0, sem_w1)

        @pl.when(sid == 0)
        def _():
            pltpu.sync_copy(t2_hbm, t2_sh)

        plsc.subcore_barrier()

        lane = lax.iota(jnp.int32, 16)
        low = lane < 8
        perm_e = (2 * lane) & 15
        perm_o = (2 * lane + 1) & 15

        def outer(io, carry):
            for b in range(_NBUF):
                i = _NBUF * io + b
                p0 = wid * per_w + i * _CHUNK
                pltpu.sync_copy(xf_hbm.at[pl.ds(2 * p0, 2 * _CHUNK)], raw_v)
                for t in range(_CHUNK // 16):
                    va = raw_v[pl.ds(32 * t, 16)]
                    vb = raw_v[pl.ds(32 * t + 16, 16)]
                    ev = jnp.where(low, _take16(va, perm_e), _take16(vb, perm_e))
                    od = jnp.where(low, _take16(va, perm_o), _take16(vb, perm_o))
                    pair_v[t // 8, pl.ds((t % 8) * 16, 16)] = ev * 10 + od

                @pl.when(io >= 1)
                def _():
                    pltpu.make_async_copy(
                        rows_v.at[b], out_hbm.at[pl.ds(0, _CHUNK)], sem_w[b]
                    ).wait()

                copies = [
                    pltpu.async_copy(
                        t2_sh.at[pair_v.at[j]],
                        rows_v.at[b, pl.ds(j * _LANES, _LANES)],
                        sem_g,
                    )
                    for j in range(_G)
                ]
                for c in copies:
                    c.wait()
                pltpu.async_copy(
                    rows_v.at[b], out_hbm.at[pl.ds(p0, _CHUNK)], sem_w[b]
                )
            return carry

        lax.fori_loop(0, steps // _NBUF, outer, 0)
        for b in range(_NBUF):
            pltpu.make_async_copy(
                rows_v.at[b], out_hbm.at[pl.ds(0, _CHUNK)], sem_w[b]
            ).wait()

    return k(xf, t2)


def kernel(x, table):
    b, s = x.shape
    v, d = table.shape
    n = b * s
    t2 = jnp.concatenate(
        [
            jnp.broadcast_to(table[:, None, :], (v, v, d)),
            jnp.broadcast_to(table[None, :, :], (v, v, d)),
        ],
        axis=-1,
    ).reshape(v * v, 2 * d)
    out = _gather_pairs(x.reshape(n), t2, n // 2)
    return out.reshape(b, s, d)


# speedup vs baseline: 1.5161x; 1.0227x over previous
import functools

import jax
import jax.numpy as jnp
from jax import lax
from jax.experimental import pallas as pl
from jax.experimental.pallas import tpu as pltpu
from jax.experimental.pallas import tpu_sc as plsc

_TAKE_DN = lax.GatherDimensionNumbers(
    offset_dims=(), collapsed_slice_dims=(0,), start_index_map=(0,)
)


def _take16(v, idx):
    return lax.gather(
        v, idx[:, None], _TAKE_DN, slice_sizes=(1,),
        mode=lax.GatherScatterMode.PROMISE_IN_BOUNDS,
    )


_S = 200
_PPR = _S // 2
_R = 4
_NBUF = 2


@functools.partial(jax.jit, static_argnames=("n_rows",))
def _gather_pairs(x2d, t2, n_rows):
    info = plsc.get_sparse_core_info()
    nw = info.num_cores * info.num_subcores
    rows_w = n_rows // nw
    steps = rows_w // _R
    n_pairs = n_rows * _PPR
    mesh = plsc.VectorSubcoreMesh(core_axis_name="c", subcore_axis_name="s")

    @functools.partial(
        pl.kernel,
        mesh=mesh,
        out_type=jax.ShapeDtypeStruct((n_pairs, 2 * 64), jnp.float32),
        scratch_types=[
            pltpu.VMEM((_R, _S), jnp.int32),
            pltpu.VMEM((_R, 128), jnp.int32),
            pltpu.VMEM((_NBUF, _R * _PPR, 2 * 64), jnp.float32),
            pltpu.VMEM_SHARED((100, 2 * 64), jnp.float32),
            pltpu.SemaphoreType.DMA,
            pltpu.SemaphoreType.DMA,
            pltpu.SemaphoreType.DMA,
        ],
    )
    def k(x_hbm, t2_hbm, out_hbm, raw_v, pair_v, rows_v,
          t2_sh, sem_g, sem_w0, sem_w1):
        sid = lax.axis_index("s")
        wid = sid * info.num_cores + lax.axis_index("c")
        sem_w = (sem_w0, sem_w1)

        @pl.when(sid == 0)
        def _():
            pltpu.sync_copy(t2_hbm, t2_sh)

        plsc.subcore_barrier()

        lane = lax.iota(jnp.int32, 16)
        low = lane < 8
        perm_e = (2 * lane) & 15
        perm_o = (2 * lane + 1) & 15
        perm_te = (8 + 2 * lane) & 15
        perm_to = (9 + 2 * lane) & 15

        def outer(io, carry):
            for b in range(_NBUF):
                i = _NBUF * io + b
                r0 = wid * rows_w + i * _R
                pltpu.sync_copy(x_hbm.at[pl.ds(r0, _R)], raw_v)
                for j in range(_R):
                    for t in range(6):
                        va = raw_v[j, pl.ds(32 * t, 16)]
                        vb = raw_v[j, pl.ds(32 * t + 16, 16)]
                        ev = jnp.where(low, _take16(va, perm_e),
                                       _take16(vb, perm_e))
                        od = jnp.where(low, _take16(va, perm_o),
                                       _take16(vb, perm_o))
                        pair_v[j, pl.ds(16 * t, 16)] = ev * 10 + od
                    vt = raw_v[j, pl.ds(_S - 16, 16)]
                    pair_v[j, pl.ds(96, 16)] = (
                        _take16(vt, perm_te) * 10 + _take16(vt, perm_to)
                    )

                @pl.when(io >= 1)
                def _():
                    pltpu.make_async_copy(
                        rows_v.at[b], out_hbm.at[pl.ds(0, _R * _PPR)], sem_w[b]
                    ).wait()

                copies = [
                    pltpu.async_copy(
                        t2_sh.at[pair_v.at[j, pl.ds(0, _PPR)]],
                        rows_v.at[b, pl.ds(j * _PPR, _PPR)],
                        sem_g,
                    )
                    for j in range(_R)
                ]
                for c in copies:
                    c.wait()
                pltpu.async_copy(
                    rows_v.at[b],
                    out_hbm.at[pl.ds(r0 * _PPR, _R * _PPR)],
                    sem_w[b],
                )
            return carry

        lax.fori_loop(0, steps // _NBUF, outer, 0)
        for b in range(_NBUF):
            pltpu.make_async_copy(
                rows_v.at[b], out_hbm.at[pl.ds(0, _R * _PPR)], sem_w[b]
            ).wait()

    return k(x2d, t2)


def kernel(x, table):
    b, s = x.shape
    v, d = table.shape
    t2 = jnp.concatenate(
        [
            jnp.broadcast_to(table[:, None, :], (v, v, d)),
            jnp.broadcast_to(table[None, :, :], (v, v, d)),
        ],
        axis=-1,
    ).reshape(v * v, 2 * d)
    out = _gather_pairs(x, t2, b)
    return out.reshape(b, s, d)


# speedup vs baseline: 1.9523x; 1.2877x over previous
import functools

import jax
import jax.numpy as jnp
from jax import lax
from jax.experimental import pallas as pl
from jax.experimental.pallas import tpu as pltpu
from jax.experimental.pallas import tpu_sc as plsc

_S = 200
_R = 1
_NBUF = 2


@functools.partial(jax.jit, static_argnames=("n_rows", "d"))
def _gather_rows(x2d, table, n_rows, d):
    info = plsc.get_sparse_core_info()
    nw = info.num_cores * info.num_subcores
    rows_w = n_rows // nw
    steps = rows_w // _R
    mesh = plsc.VectorSubcoreMesh(core_axis_name="c", subcore_axis_name="s")

    @functools.partial(
        pl.kernel,
        mesh=mesh,
        out_type=jax.ShapeDtypeStruct((n_rows, _S, d), jnp.float32),
        scratch_types=[
            pltpu.VMEM((_R, _S), jnp.int32),
            pltpu.VMEM((_NBUF, _R, _S, d), jnp.float32),
            pltpu.VMEM_SHARED((10, d), jnp.float32),
            pltpu.SemaphoreType.DMA,
            pltpu.SemaphoreType.DMA,
            pltpu.SemaphoreType.DMA,
        ],
    )
    def k(x_hbm, t_hbm, out_hbm, raw_v, rows_v, t_sh, sem_g, sem_w0, sem_w1):
        sid = lax.axis_index("s")
        wid = sid * info.num_cores + lax.axis_index("c")
        sem_w = (sem_w0, sem_w1)

        @pl.when(sid == 0)
        def _():
            pltpu.sync_copy(t_hbm, t_sh)

        plsc.subcore_barrier()

        def outer(io, carry):
            for b in range(_NBUF):
                i = _NBUF * io + b
                r0 = wid * rows_w + i * _R
                pltpu.sync_copy(x_hbm.at[pl.ds(r0, _R)], raw_v)

                @pl.when(io >= 1)
                def _():
                    pltpu.make_async_copy(
                        rows_v.at[b], out_hbm.at[pl.ds(0, _R)], sem_w[b]
                    ).wait()

                copies = []
                for j in range(_R):
                    copies.append(pltpu.async_copy(
                        t_sh.at[raw_v.at[j, pl.ds(0, 128)]],
                        rows_v.at[b, j, pl.ds(0, 128)],
                        sem_g,
                    ))
                    copies.append(pltpu.async_copy(
                        t_sh.at[raw_v.at[j, pl.ds(128, _S - 128)]],
                        rows_v.at[b, j, pl.ds(128, _S - 128)],
                        sem_g,
                    ))
                for c in copies:
                    c.wait()
                pltpu.async_copy(
                    rows_v.at[b], out_hbm.at[pl.ds(r0, _R)], sem_w[b]
                )
            return carry

        lax.fori_loop(0, steps // _NBUF, outer, 0)
        for b in range(_NBUF):
            pltpu.make_async_copy(
                rows_v.at[b], out_hbm.at[pl.ds(0, _R)], sem_w[b]
            ).wait()

    return k(x2d, table)


def kernel(x, table):
    b, s = x.shape
    v, d = table.shape
    return _gather_rows(x, table, b, d)
